# Initial kernel scaffold; baseline (speedup 1.0000x reference)
import jax
import jax.numpy as jnp
from jax.experimental import pallas as pl


def kernel(x, edge_index, W, b):
    raise NotImplementedError("write your pallas kernel here")


# speedup vs baseline: 10.8553x; 10.8553x over previous
import functools

import jax
import jax.numpy as jnp
from jax import lax
from jax.experimental import pallas as pl
from jax.experimental.pallas import tpu as pltpu
from jax.experimental.pallas import tpu_sc as plsc

N_NODES = 10000
D = 128
N_EDGES = 320000
MAX_NORM = 1e6
BOUNDARY = 1.0 - 1e-5

NC = 2
NS = 16
NW = NC * NS
CH = 80
NCHUNK = N_EDGES // (NW * CH)
PACK = 16384
STRIPE = 624
TAIL = N_NODES - NS * STRIPE
ZROWS = 16


def _artanh(v):
    v = jnp.clip(v, -1.0 + 1e-7, 1.0 - 1e-7)
    return 0.5 * jnp.log((1.0 + v) / (1.0 - v))


def _rownorm(v):
    return jnp.sqrt(jnp.sum(v * v, axis=-1, keepdims=True))


def _proj(v):
    n = jnp.clip(_rownorm(v), 1e-15, None)
    return jnp.where(n > BOUNDARY, v / n * BOUNDARY, v)


def _expmap0(u):
    n = jnp.clip(_rownorm(u), 1e-15, None)
    return jnp.tanh(n) * u / n


def _logmap0(v):
    n = jnp.clip(_rownorm(v), 1e-15, None)
    return v / n * _artanh(n)


def _hyplinear_body(x_ref, w_ref, b_ref, o_ref):
    x = x_ref[...]
    w = w_ref[...]
    b = b_ref[...]
    x_norm = jnp.clip(_rownorm(x), 1e-15, None)
    mx = lax.dot_general(x, w, (((1,), (1,)), ((), ())),
                         preferred_element_type=jnp.float32)
    mx_norm = jnp.clip(_rownorm(mx), 1e-15, None)
    res_c = jnp.tanh(mx_norm / x_norm * _artanh(x_norm)) * mx / mx_norm
    cond = jnp.max(jnp.abs(mx), axis=-1, keepdims=True) == 0.0
    res = jnp.where(cond, 0.0, res_c)
    res = _proj(res)
    hb = _proj(_expmap0(b))
    x2 = jnp.sum(res * res, axis=-1, keepdims=True)
    y2 = jnp.sum(hb * hb, axis=-1, keepdims=True)
    xy = jnp.sum(res * hb, axis=-1, keepdims=True)
    num = (1.0 + 2.0 * xy + y2) * res + (1.0 - x2) * hb
    den = 1.0 + 2.0 * xy + x2 * y2
    h = _proj(num / jnp.clip(den, 1e-15, None))
    o_ref[...] = _logmap0(h)


def _hyplinear(x, W, b):
    br = 1000
    return pl.pallas_call(
        _hyplinear_body,
        grid=(N_NODES // br,),
        in_specs=[
            pl.BlockSpec((br, D), lambda i: (i, 0)),
            pl.BlockSpec((D, D), lambda i: (0, 0)),
            pl.BlockSpec((1, D), lambda i: (0, 0)),
        ],
        out_specs=pl.BlockSpec((br, D), lambda i: (i, 0)),
        out_shape=jax.ShapeDtypeStruct((N_NODES, D), jnp.float32),
    )(x, W, b.reshape(1, D))


def _post_body(p_ref, o_ref):
    s = p_ref[0] + p_ref[1]
    s = jnp.minimum(s, MAX_NORM)
    h = _proj(_expmap0(s))
    xt = jnp.minimum(jax.nn.relu(_logmap0(h)), MAX_NORM)
    o_ref[...] = _expmap0(xt)


def _post(partials):
    br = 1000
    return pl.pallas_call(
        _post_body,
        grid=(N_NODES // br,),
        in_specs=[pl.BlockSpec((2, br, D), lambda i: (0, i, 0))],
        out_specs=pl.BlockSpec((br, D), lambda i: (i, 0)),
        out_shape=jax.ShapeDtypeStruct((N_NODES, D), jnp.float32),
    )(partials)


def _sc_agg_body(pk_hbm, tang_hbm, out_hbm,
                 pk_v, src_c, dst_c, buf_v, zbuf_v, acc_sh, gsem, ssem):
    cid = lax.axis_index("c")
    sid = lax.axis_index("s")
    wid = cid * NS + sid

    pltpu.sync_copy(pk_hbm.at[wid], pk_v)

    def zrow(i, carry):
        for c16 in range(D // 16):
            zbuf_v[i, pl.ds(c16 * 16, 16)] = jnp.zeros((16,), jnp.float32)
        return carry
    lax.fori_loop(0, ZROWS, zrow, 0)
    row0 = sid * STRIPE
    for q in range(STRIPE // ZROWS):
        pltpu.sync_copy(zbuf_v, acc_sh.at[pl.ds(row0 + q * ZROWS, ZROWS)])

    @pl.when(sid == NS - 1)
    def _():
        pltpu.sync_copy(zbuf_v, acc_sh.at[pl.ds(NS * STRIPE, TAIL)])
    plsc.subcore_barrier()

    def unpack(j, s):
        for k in range(CH // 16):
            p = pk_v[j, pl.ds(k * 16, 16)]
            src_c[s, pl.ds(k * 16, 16)] = lax.shift_right_logical(p, 14)
            dst_c[s, pl.ds(k * 16, 16)] = lax.bitwise_and(p, PACK - 1)

    def g_start(j, s):
        del j
        pltpu.async_copy(tang_hbm.at[src_c.at[s]], buf_v.at[s], gsem)

    def g_wait(j, s):
        del j
        pltpu.make_async_copy(tang_hbm.at[src_c.at[s]], buf_v.at[s], gsem).wait()

    def s_start(j, s):
        del j
        pltpu.async_copy(buf_v.at[s], acc_sh.at[dst_c.at[s]], ssem, add=True)

    def s_wait(j, s):
        del j
        pltpu.make_async_copy(buf_v.at[s], acc_sh.at[dst_c.at[s]], ssem).wait()

    unpack(0, 0)
    g_start(0, 0)

    def body(j, carry):
        s = lax.rem(j, 2)
        o = 1 - s

        @pl.when(j >= 1)
        def _():
            s_wait(j - 1, o)

        @pl.when(j + 1 < NCHUNK)
        def _():
            unpack(j + 1, o)
            g_start(j + 1, o)

        g_wait(j, s)
        s_start(j, s)
        return carry

    lax.fori_loop(0, NCHUNK, body, 0)
    s_wait(NCHUNK - 1, (NCHUNK - 1) % 2)
    plsc.subcore_barrier()

    pltpu.sync_copy(acc_sh.at[pl.ds(row0, STRIPE)],
                    out_hbm.at[cid, pl.ds(row0, STRIPE)])

    @pl.when(sid == NS - 1)
    def _():
        pltpu.sync_copy(acc_sh.at[pl.ds(NS * STRIPE, TAIL)],
                        out_hbm.at[cid, pl.ds(NS * STRIPE, TAIL)])


_sc_agg = functools.partial(
    pl.kernel,
    out_type=jax.ShapeDtypeStruct((NC, N_NODES, D), jnp.float32),
    mesh=plsc.VectorSubcoreMesh(core_axis_name="c", subcore_axis_name="s"),
    scratch_types=[
        pltpu.VMEM((NCHUNK, CH), jnp.int32),
        pltpu.VMEM((2, CH), jnp.int32),
        pltpu.VMEM((2, CH), jnp.int32),
        pltpu.VMEM((2, CH, D), jnp.float32),
        pltpu.VMEM((ZROWS, D), jnp.float32),
        pltpu.VMEM_SHARED((N_NODES, D), jnp.float32),
        pltpu.SemaphoreType.DMA,
        pltpu.SemaphoreType.DMA,
    ],
)(_sc_agg_body)


def kernel(x, edge_index, W, b):
    xt = _hyplinear(x, W, b)
    ei = edge_index.astype(jnp.int32)
    packed = (ei[0] * PACK + ei[1]).reshape(NW, NCHUNK, CH)
    partials = _sc_agg(packed, xt)
    return _post(partials)
